# baseline (speedup 1.0000x reference)
import functools
import math

import jax
import jax.numpy as jnp
import numpy as np
from jax import lax
from jax.experimental import pallas as pl
from jax.experimental.pallas import tpu as pltpu
from jax.experimental.pallas import tpu_sc as plsc

_H = 512
_W = 512
_NPIX = _H * _W
_NPL = 12
_NPOS = 300
_NNEG = 300
_NRING = _NPOS * _NNEG
_MARGIN = 0.5
_NCH = 32
_CH = _NPIX // _NCH
_NSUB = 16
_KPAD = 384


def _tf2x32_core(k1, k2, x0, x1):
    rot0 = (13, 15, 26, 6)
    rot1 = (17, 29, 16, 24)
    ks = [np.uint32(k1), np.uint32(k2),
          np.uint32(k1) ^ np.uint32(k2) ^ np.uint32(0x1BD11BDA)]
    x = [x0.astype(np.uint32) + ks[0], x1.astype(np.uint32) + ks[1]]

    def rnd(x, r):
        x0 = x[0] + x[1]
        x1 = (x[1] << np.uint32(r)) | (x[1] >> np.uint32(32 - r))
        return [x0, x0 ^ x1]

    for rots, a, b, i in ((rot0, 1, 2, 1), (rot1, 2, 0, 2), (rot0, 0, 1, 3),
                          (rot1, 1, 2, 4), (rot0, 2, 0, 5)):
        for r in rots:
            x = rnd(x, r)
        x = [x[0] + ks[a], x[1] + ks[b] + np.uint32(i)]
    return x[0], x[1]


def _np_split(key, num):
    cnt = np.arange(num, dtype=np.uint64)
    hi = (cnt >> np.uint64(32)).astype(np.uint32)
    lo = cnt.astype(np.uint32)
    b1, b2 = _tf2x32_core(key[0], key[1], hi, lo)
    return np.stack([b1, b2], axis=1)


def _np_random_bits(key, shape):
    n = int(np.prod(shape))
    cnt = np.arange(n, dtype=np.uint64)
    hi = (cnt >> np.uint64(32)).astype(np.uint32)
    lo = cnt.astype(np.uint32)
    b1, b2 = _tf2x32_core(key[0], key[1], hi, lo)
    return (b1 ^ b2).reshape(shape)


def _np_randint(key, shape, minval, maxval):
    k1, k2 = _np_split(key, 2)
    higher = _np_random_bits(k1, shape)
    lower = _np_random_bits(k2, shape)
    span = np.uint32(maxval - minval)
    mult = np.uint32(2 ** 16) % span
    mult = np.uint32(mult * mult) % span
    off = ((higher % span) * mult + lower % span) % span
    return (np.int32(minval) + off.astype(np.int32)).astype(np.int32)


def _np_uniform(key, shape):
    bits = _np_random_bits(key, shape)
    fb = (bits >> np.uint32(9)) | np.uint32(0x3F800000)
    return fb.view(np.float32) - np.float32(1.0)


def _ring_coords(key, feats, height):
    rep = np.tile(feats, (_NNEG, 1)).astype(np.float32)
    n = rep.shape[0]
    k1, k2 = _np_split(key, 2)
    radius = _np_uniform(k1, (1, n)) * np.float32(25.0) + np.float32(5.0)
    angle = _np_uniform(k2, (1, n)) * np.float32(2.0 * math.pi)
    x_off = radius * np.cos(angle, dtype=np.float32)
    y_off = radius * np.sin(angle, dtype=np.float32)
    nonmatch_x = rep[:, 0] + x_off
    nonmatch_y = rep[:, 1] + y_off
    nonmatch = np.stack([nonmatch_x, nonmatch_y], axis=1).squeeze()
    return np.clip(nonmatch, 0, height - 1).astype(np.int32).T


def _build_plan():
    fkey = np.array([0, 1234], dtype=np.uint32)
    k_a, k_p, k_r1, k_r2 = _np_split(fkey, 4)
    fa = _np_randint(k_a, (_NPOS, 2), 0, _H)
    fp = _np_randint(k_p, (_NPOS, 2), 0, _H)
    r1 = _ring_coords(k_r1, fa, _H)
    r2 = _ring_coords(k_r2, fp, _H)

    ia = (fa[:, 0] * _W + fa[:, 1]).astype(np.int64)
    ip = (fp[:, 0] * _W + fp[:, 1]).astype(np.int64)
    kk = np.arange(_NRING, dtype=np.int64) % _NPOS

    def buckets(ir):
        band = ir // _CH
        sub = band % _NSUB
        slot = band // _NSUB
        rel = slot * _CH + ir % _CH
        packed = (1 << 25) | (rel << 9) | kk
        return [packed[sub == s].astype(np.int64) for s in range(_NSUB)]

    b1 = buckets(r1[:, 0].astype(np.int64) * _W + r1[:, 1])
    b2 = buckets(r2[:, 0].astype(np.int64) * _W + r2[:, 1])
    bl = max(max(len(x) for x in b1), max(len(x) for x in b2))
    bl = ((bl + 15) // 16) * 16
    pk = np.zeros((2, _NSUB, bl), dtype=np.int32)
    for s in range(_NSUB):
        pk[0, s, : len(b1[s])] = b1[s]
        pk[1, s, : len(b2[s])] = b2[s]

    lens = np.zeros((2 * _NSUB,), dtype=np.int32)
    for s in range(_NSUB):
        lens[s] = (len(b1[s]) + 15) // 16
        lens[_NSUB + s] = (len(b2[s]) + 15) // 16

    ia304 = np.zeros((304,), dtype=np.int64)
    ia304[:_NPOS] = ia
    ip304 = np.zeros((304,), dtype=np.int64)
    ip304[:_NPOS] = ip
    return pk.reshape(-1), lens, ia304, ip304, bl


_PK_NP, _LENS_NP, _IA304_NP, _IP304_NP, _BL = _build_plan()


@functools.lru_cache(maxsize=1)
def _make_sc_kernel():
    mesh = plsc.VectorSubcoreMesh(core_axis_name="c", subcore_axis_name="s")
    f32 = jnp.float32
    i32 = jnp.int32

    @functools.partial(
        pl.kernel,
        mesh=mesh,
        compiler_params=pltpu.CompilerParams(
            needs_layout_passes=False, use_tc_tiling_on_sc=True),
        out_type=[
            jax.ShapeDtypeStruct((2 * _NSUB * 16,), f32),
        ],
        scratch_types=[
            pltpu.VMEM((64, 512), f32),
            pltpu.VMEM((_BL,), i32),
            pltpu.VMEM((2 * _NPL * 304,), f32),
            pltpu.VMEM((2 * _NSUB,), i32),
            pltpu.VMEM((16,), f32),
            pltpu.SemaphoreType.DMA,
        ],
    )
    def sc_main(i1r, i2r, pk, lens, sml, outd,
                chunkv, bucket, smalls, lensv, obuf, semc):
        c = lax.axis_index("c")
        s = lax.axis_index("s")
        lane = lax.iota(i32, 16)

        pltpu.sync_copy(pk.at[pl.ds((c * _NSUB + s) * _BL, _BL)], bucket)
        pltpu.sync_copy(sml, smalls)
        pltpu.sync_copy(lens, lensv)
        trip = jnp.sum(jnp.where(lane == s,
                                 lensv[pl.ds(c * _NSUB, _NSUB)], 0))

        def run(myf):
            def issue(q, par):
                for t in range(2):
                    ch = s + _NSUB * t
                    pltpu.async_copy(
                        myf.at[q, pl.ds(ch * 16, 16), :],
                        chunkv.at[pl.ds((par * 2 + t) * 16, 16), :],
                        semc,
                    )

            def wait_plane(par):
                for t in range(2):
                    pltpu.make_async_copy(
                        myf.at[0, pl.ds(0, 16), :],
                        chunkv.at[pl.ds((par * 2 + t) * 16, 16), :],
                        semc,
                    ).wait()

            issue(0, 0)

            def plane_body(p, outvec):
                par = p & 1
                wait_plane(par)

                @pl.when(p < _NPL - 1)
                def _():
                    issue(p + 1, 1 - par)

                roff = jnp.full((16,), par * 32, i32)
                soff = jnp.full((16,), (c * _NPL + p) * 304, i32)

                @plsc.parallel_loop(0, trip * 16, 16, unroll=4,
                                    carry=jnp.zeros((16,), f32))
                def acc(i, acc):
                    pkv = bucket[pl.ds(i, 16)]
                    validf = (pkv >> 25).astype(f32)
                    row = ((pkv >> 18) & 0x1F) + roff
                    col = (pkv >> 9) & 0x1FF
                    kv = (pkv & 0x1FF) + soff
                    v = plsc.load_gather(chunkv, [row, col])
                    a = plsc.load_gather(smalls, [kv])
                    d = a - v
                    return acc + d * d * validf

                ps = jnp.sum(acc)
                return outvec + jnp.where(lane == p, ps, 0.0)

            outvec = lax.fori_loop(0, _NPL, plane_body,
                                   jnp.zeros((16,), f32))
            obuf[...] = outvec
            pltpu.sync_copy(obuf, outd.at[pl.ds((c * _NSUB + s) * 16, 16)])

        @pl.when(c == 0)
        def _():
            run(i1r)

        @pl.when(c == 1)
        def _():
            run(i2r)

    return sc_main


def _epilogue_body(d_ref, a_ref, p_ref, o_ref):
    d = d_ref[...]
    av = a_ref[...]
    pv = p_ref[...]
    d2 = jnp.sum(d[0:16, :], axis=0, keepdims=True)
    d1 = jnp.sum(d[16:32, :], axis=0, keepdims=True)
    kmask = lax.broadcasted_iota(jnp.int32, (1, 304), 1) < _NPOS
    dd = av - pv
    s1 = jnp.sum(jnp.where(kmask, dd * dd, jnp.zeros_like(dd)),
                 axis=1, keepdims=True)
    lm = jnp.sum(jnp.sqrt(s1 + 1e-7), axis=0, keepdims=True)
    msk = lax.broadcasted_iota(jnp.int32, (1, 16), 1) < _NPL
    zero = jnp.zeros((1, 16), jnp.float32)
    l1 = jnp.where(msk, jnp.maximum(0.0, _MARGIN - jnp.sqrt(d1 + 1e-7)), zero)
    l2 = jnp.where(msk, jnp.maximum(0.0, _MARGIN - jnp.sqrt(d2 + 1e-7)), zero)
    tot = (lm + jnp.sum(l1 + l2, axis=1, keepdims=True)) / float(_NPL)
    o_ref[...] = tot


def kernel(img1, img1_out, img2, img2_out):
    del img1, img2
    i1r = img1_out.reshape(_NPL, _H, _W)
    i2r = img2_out.reshape(_NPL, _H, _W)
    avals = i1r.reshape(_NPL, _NPIX)[:, _IA304_NP]
    pvals = i2r.reshape(_NPL, _NPIX)[:, _IP304_NP]
    sml = jnp.concatenate([pvals.reshape(-1), avals.reshape(-1)])
    pk = jnp.asarray(_PK_NP)
    lens = jnp.asarray(_LENS_NP)
    (outd,) = _make_sc_kernel()(i1r, i2r, pk, lens, sml)
    res = pl.pallas_call(
        _epilogue_body,
        out_shape=jax.ShapeDtypeStruct((1, 1), jnp.float32),
    )(outd.reshape(2 * _NSUB, 16), avals, pvals)
    return res[0, 0]


# speedup vs baseline: 1.7740x; 1.7740x over previous
import functools
import math

import jax
import jax.numpy as jnp
import numpy as np
from jax import lax
from jax.experimental import pallas as pl
from jax.experimental.pallas import tpu as pltpu
from jax.experimental.pallas import tpu_sc as plsc

_H = 512
_W = 512
_NPIX = _H * _W
_NPL = 12
_NPOS = 300
_NNEG = 300
_NRING = _NPOS * _NNEG
_MARGIN = 0.5
_NCH = 32
_CH = _NPIX // _NCH
_NSUB = 16
_KPAD = 384


def _tf2x32_core(k1, k2, x0, x1):
    rot0 = (13, 15, 26, 6)
    rot1 = (17, 29, 16, 24)
    ks = [np.uint32(k1), np.uint32(k2),
          np.uint32(k1) ^ np.uint32(k2) ^ np.uint32(0x1BD11BDA)]
    x = [x0.astype(np.uint32) + ks[0], x1.astype(np.uint32) + ks[1]]

    def rnd(x, r):
        x0 = x[0] + x[1]
        x1 = (x[1] << np.uint32(r)) | (x[1] >> np.uint32(32 - r))
        return [x0, x0 ^ x1]

    for rots, a, b, i in ((rot0, 1, 2, 1), (rot1, 2, 0, 2), (rot0, 0, 1, 3),
                          (rot1, 1, 2, 4), (rot0, 2, 0, 5)):
        for r in rots:
            x = rnd(x, r)
        x = [x[0] + ks[a], x[1] + ks[b] + np.uint32(i)]
    return x[0], x[1]


def _np_split(key, num):
    cnt = np.arange(num, dtype=np.uint64)
    hi = (cnt >> np.uint64(32)).astype(np.uint32)
    lo = cnt.astype(np.uint32)
    b1, b2 = _tf2x32_core(key[0], key[1], hi, lo)
    return np.stack([b1, b2], axis=1)


def _np_random_bits(key, shape):
    n = int(np.prod(shape))
    cnt = np.arange(n, dtype=np.uint64)
    hi = (cnt >> np.uint64(32)).astype(np.uint32)
    lo = cnt.astype(np.uint32)
    b1, b2 = _tf2x32_core(key[0], key[1], hi, lo)
    return (b1 ^ b2).reshape(shape)


def _np_randint(key, shape, minval, maxval):
    k1, k2 = _np_split(key, 2)
    higher = _np_random_bits(k1, shape)
    lower = _np_random_bits(k2, shape)
    span = np.uint32(maxval - minval)
    mult = np.uint32(2 ** 16) % span
    mult = np.uint32(mult * mult) % span
    off = ((higher % span) * mult + lower % span) % span
    return (np.int32(minval) + off.astype(np.int32)).astype(np.int32)


def _np_uniform(key, shape):
    bits = _np_random_bits(key, shape)
    fb = (bits >> np.uint32(9)) | np.uint32(0x3F800000)
    return fb.view(np.float32) - np.float32(1.0)


def _ring_coords(key, feats, height):
    rep = np.tile(feats, (_NNEG, 1)).astype(np.float32)
    n = rep.shape[0]
    k1, k2 = _np_split(key, 2)
    radius = _np_uniform(k1, (1, n)) * np.float32(25.0) + np.float32(5.0)
    angle = _np_uniform(k2, (1, n)) * np.float32(2.0 * math.pi)
    x_off = radius * np.cos(angle, dtype=np.float32)
    y_off = radius * np.sin(angle, dtype=np.float32)
    nonmatch_x = rep[:, 0] + x_off
    nonmatch_y = rep[:, 1] + y_off
    nonmatch = np.stack([nonmatch_x, nonmatch_y], axis=1).squeeze()
    return np.clip(nonmatch, 0, height - 1).astype(np.int32).T


def _build_plan():
    fkey = np.array([0, 1234], dtype=np.uint32)
    k_a, k_p, k_r1, k_r2 = _np_split(fkey, 4)
    fa = _np_randint(k_a, (_NPOS, 2), 0, _H)
    fp = _np_randint(k_p, (_NPOS, 2), 0, _H)
    r1 = _ring_coords(k_r1, fa, _H)
    r2 = _ring_coords(k_r2, fp, _H)

    ia = (fa[:, 0] * _W + fa[:, 1]).astype(np.int64)
    ip = (fp[:, 0] * _W + fp[:, 1]).astype(np.int64)
    kk = np.arange(_NRING, dtype=np.int64) % _NPOS

    def buckets(ir):
        band = ir // _CH
        sub = band % _NSUB
        slot = band // _NSUB
        rel = slot * _CH + ir % _CH
        packed = (1 << 25) | (rel << 9) | kk
        return [packed[sub == s].astype(np.int64) for s in range(_NSUB)]

    b1 = buckets(r1[:, 0].astype(np.int64) * _W + r1[:, 1])
    b2 = buckets(r2[:, 0].astype(np.int64) * _W + r2[:, 1])
    bl = max(max(len(x) for x in b1), max(len(x) for x in b2))
    bl = ((bl + 15) // 16) * 16
    pk = np.zeros((2, _NSUB, bl), dtype=np.int32)
    for s in range(_NSUB):
        pk[0, s, : len(b1[s])] = b1[s]
        pk[1, s, : len(b2[s])] = b2[s]

    lens = np.zeros((2 * _NSUB,), dtype=np.int32)
    for s in range(_NSUB):
        lens[s] = (len(b1[s]) + 15) // 16
        lens[_NSUB + s] = (len(b2[s]) + 15) // 16

    fa304 = np.zeros((304, 2), dtype=np.int32)
    fa304[:_NPOS] = fa
    fp304 = np.zeros((304, 2), dtype=np.int32)
    fp304[:_NPOS] = fp
    return pk.reshape(-1), lens, fa304, fp304, bl


_PK_NP, _LENS_NP, _FA304_NP, _FP304_NP, _BL = _build_plan()


@functools.lru_cache(maxsize=1)
def _make_sc_kernel():
    mesh = plsc.VectorSubcoreMesh(core_axis_name="c", subcore_axis_name="s")
    f32 = jnp.float32
    i32 = jnp.int32

    @functools.partial(
        pl.kernel,
        mesh=mesh,
        compiler_params=pltpu.CompilerParams(
            needs_layout_passes=False, use_tc_tiling_on_sc=True),
        out_type=[
            jax.ShapeDtypeStruct((2 * _NSUB * 16,), f32),
        ],
        scratch_types=[
            pltpu.VMEM((64, 512), f32),
            pltpu.VMEM((_BL,), i32),
            pltpu.VMEM((2 * _NPL * 304,), f32),
            pltpu.VMEM((2 * _NSUB,), i32),
            pltpu.VMEM((16,), f32),
            pltpu.SemaphoreType.DMA,
        ],
    )
    def sc_main(i1r, i2r, pk, lens, sml, outd,
                chunkv, bucket, smalls, lensv, obuf, semc):
        c = lax.axis_index("c")
        s = lax.axis_index("s")
        lane = lax.iota(i32, 16)

        pltpu.sync_copy(pk.at[pl.ds((c * _NSUB + s) * _BL, _BL)], bucket)
        pltpu.sync_copy(sml, smalls)
        pltpu.sync_copy(lens, lensv)
        trip = jnp.sum(jnp.where(lane == s,
                                 lensv[pl.ds(c * _NSUB, _NSUB)], 0))

        def run(myf):
            def issue(q, par):
                for t in range(2):
                    ch = s + _NSUB * t
                    pltpu.async_copy(
                        myf.at[q, pl.ds(ch * 16, 16), :],
                        chunkv.at[pl.ds((par * 2 + t) * 16, 16), :],
                        semc,
                    )

            def wait_plane(par):
                for t in range(2):
                    pltpu.make_async_copy(
                        myf.at[0, pl.ds(0, 16), :],
                        chunkv.at[pl.ds((par * 2 + t) * 16, 16), :],
                        semc,
                    ).wait()

            issue(0, 0)

            def plane_body(p, outvec):
                par = p & 1
                wait_plane(par)

                @pl.when(p < _NPL - 1)
                def _():
                    issue(p + 1, 1 - par)

                roff = jnp.full((16,), par * 32, i32)
                soff = jnp.full((16,), (c * _NPL + p) * 304, i32)

                @plsc.parallel_loop(0, trip * 16, 16, unroll=4,
                                    carry=jnp.zeros((16,), f32))
                def acc(i, acc):
                    pkv = bucket[pl.ds(i, 16)]
                    validf = (pkv >> 25).astype(f32)
                    row = ((pkv >> 18) & 0x1F) + roff
                    col = (pkv >> 9) & 0x1FF
                    kv = (pkv & 0x1FF) + soff
                    v = plsc.load_gather(chunkv, [row, col])
                    a = plsc.load_gather(smalls, [kv])
                    d = a - v
                    return acc + d * d * validf

                ps = jnp.sum(acc)
                return outvec + jnp.where(lane == p, ps, 0.0)

            outvec = lax.fori_loop(0, _NPL, plane_body,
                                   jnp.zeros((16,), f32))
            obuf[...] = outvec
            pltpu.sync_copy(obuf, outd.at[pl.ds((c * _NSUB + s) * 16, 16)])

        @pl.when(c == 0)
        def _():
            run(i1r)

        @pl.when(c == 1)
        def _():
            run(i2r)

    return sc_main


def _epilogue_body(d_ref, a_ref, p_ref, o_ref):
    d = d_ref[...]
    av = a_ref[...]
    pv = p_ref[...]
    d2 = jnp.sum(d[0:16, :], axis=0, keepdims=True)
    d1 = jnp.sum(d[16:32, :], axis=0, keepdims=True)
    kmask = lax.broadcasted_iota(jnp.int32, (1, 304), 1) < _NPOS
    dd = av - pv
    s1 = jnp.sum(jnp.where(kmask, dd * dd, jnp.zeros_like(dd)),
                 axis=1, keepdims=True)
    lm = jnp.sum(jnp.sqrt(s1 + 1e-7), axis=0, keepdims=True)
    msk = lax.broadcasted_iota(jnp.int32, (1, 16), 1) < _NPL
    zero = jnp.zeros((1, 16), jnp.float32)
    l1 = jnp.where(msk, jnp.maximum(0.0, _MARGIN - jnp.sqrt(d1 + 1e-7)), zero)
    l2 = jnp.where(msk, jnp.maximum(0.0, _MARGIN - jnp.sqrt(d2 + 1e-7)), zero)
    tot = (lm + jnp.sum(l1 + l2, axis=1, keepdims=True)) / float(_NPL)
    o_ref[...] = tot


def kernel(img1, img1_out, img2, img2_out):
    del img1, img2
    i1r = img1_out.reshape(_NPL, _H, _W)
    i2r = img2_out.reshape(_NPL, _H, _W)
    avals = img1_out[:, :, _FA304_NP[:, 0], _FA304_NP[:, 1]].reshape(
        _NPL, 304)
    pvals = img2_out[:, :, _FP304_NP[:, 0], _FP304_NP[:, 1]].reshape(
        _NPL, 304)
    sml = jnp.concatenate([pvals.reshape(-1), avals.reshape(-1)])
    pk = jnp.asarray(_PK_NP)
    lens = jnp.asarray(_LENS_NP)
    (outd,) = _make_sc_kernel()(i1r, i2r, pk, lens, sml)
    res = pl.pallas_call(
        _epilogue_body,
        out_shape=jax.ShapeDtypeStruct((1, 1), jnp.float32),
    )(outd.reshape(2 * _NSUB, 16), avals, pvals)
    return res[0, 0]


# speedup vs baseline: 2.0561x; 1.1590x over previous
import functools
import math

import jax
import jax.numpy as jnp
import numpy as np
from jax import lax
from jax.experimental import pallas as pl
from jax.experimental.pallas import tpu as pltpu
from jax.experimental.pallas import tpu_sc as plsc

_H = 512
_W = 512
_NPIX = _H * _W
_NPL = 12
_NPOS = 300
_NNEG = 300
_NRING = _NPOS * _NNEG
_MARGIN = 0.5
_NCH = 32
_CH = _NPIX // _NCH
_NSUB = 16
_KPAD = 384


def _tf2x32_core(k1, k2, x0, x1):
    rot0 = (13, 15, 26, 6)
    rot1 = (17, 29, 16, 24)
    ks = [np.uint32(k1), np.uint32(k2),
          np.uint32(k1) ^ np.uint32(k2) ^ np.uint32(0x1BD11BDA)]
    x = [x0.astype(np.uint32) + ks[0], x1.astype(np.uint32) + ks[1]]

    def rnd(x, r):
        x0 = x[0] + x[1]
        x1 = (x[1] << np.uint32(r)) | (x[1] >> np.uint32(32 - r))
        return [x0, x0 ^ x1]

    for rots, a, b, i in ((rot0, 1, 2, 1), (rot1, 2, 0, 2), (rot0, 0, 1, 3),
                          (rot1, 1, 2, 4), (rot0, 2, 0, 5)):
        for r in rots:
            x = rnd(x, r)
        x = [x[0] + ks[a], x[1] + ks[b] + np.uint32(i)]
    return x[0], x[1]


def _np_split(key, num):
    cnt = np.arange(num, dtype=np.uint64)
    hi = (cnt >> np.uint64(32)).astype(np.uint32)
    lo = cnt.astype(np.uint32)
    b1, b2 = _tf2x32_core(key[0], key[1], hi, lo)
    return np.stack([b1, b2], axis=1)


def _np_random_bits(key, shape):
    n = int(np.prod(shape))
    cnt = np.arange(n, dtype=np.uint64)
    hi = (cnt >> np.uint64(32)).astype(np.uint32)
    lo = cnt.astype(np.uint32)
    b1, b2 = _tf2x32_core(key[0], key[1], hi, lo)
    return (b1 ^ b2).reshape(shape)


def _np_randint(key, shape, minval, maxval):
    k1, k2 = _np_split(key, 2)
    higher = _np_random_bits(k1, shape)
    lower = _np_random_bits(k2, shape)
    span = np.uint32(maxval - minval)
    mult = np.uint32(2 ** 16) % span
    mult = np.uint32(mult * mult) % span
    off = ((higher % span) * mult + lower % span) % span
    return (np.int32(minval) + off.astype(np.int32)).astype(np.int32)


def _np_uniform(key, shape):
    bits = _np_random_bits(key, shape)
    fb = (bits >> np.uint32(9)) | np.uint32(0x3F800000)
    return fb.view(np.float32) - np.float32(1.0)


def _ring_coords(key, feats, height):
    rep = np.tile(feats, (_NNEG, 1)).astype(np.float32)
    n = rep.shape[0]
    k1, k2 = _np_split(key, 2)
    radius = _np_uniform(k1, (1, n)) * np.float32(25.0) + np.float32(5.0)
    angle = _np_uniform(k2, (1, n)) * np.float32(2.0 * math.pi)
    x_off = radius * np.cos(angle, dtype=np.float32)
    y_off = radius * np.sin(angle, dtype=np.float32)
    nonmatch_x = rep[:, 0] + x_off
    nonmatch_y = rep[:, 1] + y_off
    nonmatch = np.stack([nonmatch_x, nonmatch_y], axis=1).squeeze()
    return np.clip(nonmatch, 0, height - 1).astype(np.int32).T


def _build_plan():
    fkey = np.array([0, 1234], dtype=np.uint32)
    k_a, k_p, k_r1, k_r2 = _np_split(fkey, 4)
    fa = _np_randint(k_a, (_NPOS, 2), 0, _H)
    fp = _np_randint(k_p, (_NPOS, 2), 0, _H)
    r1 = _ring_coords(k_r1, fa, _H)
    r2 = _ring_coords(k_r2, fp, _H)

    ia = (fa[:, 0] * _W + fa[:, 1]).astype(np.int64)
    ip = (fp[:, 0] * _W + fp[:, 1]).astype(np.int64)
    kk = np.arange(_NRING, dtype=np.int64) % _NPOS

    def buckets(ir):
        band = ir // _CH
        sizes = np.bincount(band, minlength=_NCH)
        order = np.argsort(sizes)
        q0s = order[:_NSUB].astype(np.int32)
        q1s = order[_NSUB:][::-1].astype(np.int32)
        lists = []
        for s in range(_NSUB):
            e0 = np.nonzero(band == q0s[s])[0]
            e1 = np.nonzero(band == q1s[s])[0]
            rel = np.concatenate([ir[e0] % _CH, _CH + ir[e1] % _CH])
            kkv = np.concatenate([kk[e0], kk[e1]])
            lists.append(((1 << 25) | (rel << 9) | kkv).astype(np.int64))
        return lists, q0s, q1s

    b1, q10, q11 = buckets(r1[:, 0].astype(np.int64) * _W + r1[:, 1])
    b2, q20, q21 = buckets(r2[:, 0].astype(np.int64) * _W + r2[:, 1])
    bl = max(max(len(x) for x in b1), max(len(x) for x in b2))
    bl = ((bl + 15) // 16) * 16
    pk = np.zeros((2, _NSUB, bl), dtype=np.int32)
    for s in range(_NSUB):
        pk[0, s, : len(b1[s])] = b1[s]
        pk[1, s, : len(b2[s])] = b2[s]
    chtab = np.concatenate([q10, q11, q20, q21]).astype(np.int32)

    def idx_table(base_idx):
        t = np.zeros((_NPL, _KPAD), dtype=np.int64)
        t[:, : _NPOS] = base_idx[None, :]
        t += (np.arange(_NPL, dtype=np.int64) * _NPIX)[:, None]
        return t

    gix = np.concatenate([idx_table(ip), idx_table(ia)], axis=0)
    gix = gix.reshape(2 * _NPL * 3, 128).astype(np.int32)
    lens = np.zeros((2 * _NSUB,), dtype=np.int32)
    for s in range(_NSUB):
        lens[s] = (len(b1[s]) + 15) // 16
        lens[_NSUB + s] = (len(b2[s]) + 15) // 16
    return pk.reshape(-1), gix, lens, chtab, bl


_PK_NP, _GIX_NP, _LENS_NP, _CHTAB_NP, _BL = _build_plan()


@functools.lru_cache(maxsize=1)
def _make_sc_kernel():
    mesh = plsc.VectorSubcoreMesh(core_axis_name="c", subcore_axis_name="s")
    f32 = jnp.float32
    i32 = jnp.int32

    @functools.partial(
        pl.kernel,
        mesh=mesh,
        compiler_params=pltpu.CompilerParams(needs_layout_passes=False),
        out_type=[
            jax.ShapeDtypeStruct((48,), f32),
        ],
        scratch_types=[
            pltpu.VMEM((2 * 2 * _CH,), f32),
            pltpu.VMEM((_BL,), i32),
            pltpu.VMEM((2 * _NPL * 3, 128), i32),
            pltpu.VMEM((2 * _NPL * _KPAD,), f32),
            pltpu.VMEM((128,), f32),
            pltpu.VMEM((2 * _NSUB,), i32),
            pltpu.VMEM((4 * _NSUB,), i32),
            pltpu.VMEM((16,), f32),
            pltpu.VMEM((16 * 16,), f32),
            pltpu.VMEM((16 * 16,), f32),
            pltpu.VMEM_SHARED((2 * _NPL * _KPAD,), f32),
            pltpu.VMEM_SHARED((16 * 16,), f32),
            pltpu.VMEM_SHARED((16 * 16,), f32),
            pltpu.SemaphoreType.DMA,
            pltpu.SemaphoreType.DMA,
        ],
    )
    def sc_main(i1f, i2f, pk, gix, lens, chtab, outd,
                chunkv, bucket, gixv, smalls, stage, lensv, chv, obuf,
                redd, reds, shared, sharedd, shareds, semc, semg):
        c = lax.axis_index("c")
        s = lax.axis_index("s")
        lane = lax.iota(i32, 16)
        nrows = 2 * _NPL * 3

        pltpu.sync_copy(pk.at[pl.ds((c * _NSUB + s) * _BL, _BL)], bucket)
        pltpu.sync_copy(gix, gixv)
        pltpu.sync_copy(lens, lensv)
        pltpu.sync_copy(chtab, chv)
        trip = jnp.sum(jnp.where(lane == s,
                                 lensv[pl.ds(c * _NSUB, _NSUB)], 0))
        ch0 = jnp.sum(jnp.where(lane == s,
                                chv[pl.ds(c * 2 * _NSUB, _NSUB)], 0))
        ch1 = jnp.sum(jnp.where(lane == s,
                                chv[pl.ds(c * 2 * _NSUB + _NSUB, _NSUB)], 0))

        for t in range((nrows + _NSUB - 1) // _NSUB):
            r = s + t * _NSUB

            @pl.when(jnp.logical_and(r < nrows, r < _NPL * 3))
            def _():
                pltpu.async_copy(i2f.at[gixv.at[r]], stage, semg).wait()
                pltpu.sync_copy(stage, shared.at[pl.ds(r * 128, 128)])

            @pl.when(jnp.logical_and(r < nrows, r >= _NPL * 3))
            def _():
                pltpu.async_copy(i1f.at[gixv.at[r]], stage, semg).wait()
                pltpu.sync_copy(stage, shared.at[pl.ds(r * 128, 128)])

        plsc.subcore_barrier()
        pltpu.sync_copy(shared, smalls)

        obuf[...] = jnp.zeros((16,), f32)

        @pl.when(jnp.logical_and(c == 0, s < _NPL))
        def _():
            acc = jnp.zeros((16,), f32)
            for i in range(_KPAD // 16):
                kl = lane + i * 16
                d = (smalls[pl.ds((_NPL + s) * _KPAD + i * 16, 16)]
                     - smalls[pl.ds(s * _KPAD + i * 16, 16)])
                acc = acc + jnp.where(kl < _NPOS, d * d, 0.0)
            obuf[...] = jnp.where(lane == s, jnp.sum(acc), 0.0)

        pltpu.sync_copy(obuf, shareds.at[pl.ds(s * 16, 16)])

        def run(myf):
            def issue(q, par):
                for t, cht in ((0, ch0), (1, ch1)):
                    pltpu.async_copy(
                        myf.at[pl.ds(q * _NPIX + cht * _CH, _CH)],
                        chunkv.at[pl.ds((par * 2 + t) * _CH, _CH)],
                        semc,
                    )

            def wait_plane(par):
                for t in range(2):
                    pltpu.make_async_copy(
                        myf.at[pl.ds(0, _CH)],
                        chunkv.at[pl.ds((par * 2 + t) * _CH, _CH)],
                        semc,
                    ).wait()

            issue(0, 0)

            def plane_body(p, outvec):
                par = p & 1
                wait_plane(par)

                @pl.when(p < _NPL - 1)
                def _():
                    issue(p + 1, 1 - par)

                coff = jnp.full((16,), par * 2 * _CH, i32)
                soff = jnp.full((16,), (c * _NPL + p) * _KPAD, i32)

                @plsc.parallel_loop(0, trip * 16, 16, unroll=4,
                                    carry=jnp.zeros((16,), f32))
                def acc(i, acc):
                    pkv = bucket[pl.ds(i, 16)]
                    validf = (pkv >> 25).astype(f32)
                    rel = ((pkv >> 9) & 0x3FFF) + coff
                    kv = (pkv & 0x1FF) + soff
                    v = plsc.load_gather(chunkv, [rel])
                    a = plsc.load_gather(smalls, [kv])
                    d = a - v
                    return acc + d * d * validf

                ps = jnp.sum(acc)
                return outvec + jnp.where(lane == p, ps, 0.0)

            outvec = lax.fori_loop(0, _NPL, plane_body,
                                   jnp.zeros((16,), f32))
            obuf[...] = outvec

        @pl.when(c == 0)
        def _():
            run(i1f)

        @pl.when(c == 1)
        def _():
            run(i2f)

        pltpu.sync_copy(obuf, sharedd.at[pl.ds(s * 16, 16)])
        plsc.subcore_barrier()

        @pl.when(s == 0)
        def _():
            pltpu.sync_copy(sharedd, redd)
            pltpu.sync_copy(shareds, reds)
            accd = jnp.zeros((16,), f32)
            accs = jnp.zeros((16,), f32)
            for i in range(_NSUB):
                accd = accd + redd[pl.ds(i * 16, 16)]
                accs = accs + reds[pl.ds(i * 16, 16)]
            obuf[...] = accd
            pltpu.sync_copy(obuf, outd.at[pl.ds(c * 16, 16)])

            @pl.when(c == 0)
            def _():
                obuf[...] = accs
                pltpu.sync_copy(obuf, outd.at[pl.ds(32, 16)])

    return sc_main


def _epilogue_body(d_ref, o_ref):
    x = d_ref[...]
    d2 = x[0:16]
    d1 = x[16:32]
    s1 = x[32:48]
    msk = lax.broadcasted_iota(jnp.int32, (16,), 0) < _NPL
    zero = jnp.zeros((16,), jnp.float32)
    lm = jnp.where(msk, jnp.sqrt(s1 + 1e-7), zero)
    l1 = jnp.where(msk, jnp.maximum(0.0, _MARGIN - jnp.sqrt(d1 + 1e-7)), zero)
    l2 = jnp.where(msk, jnp.maximum(0.0, _MARGIN - jnp.sqrt(d2 + 1e-7)), zero)
    o_ref[...] = jnp.sum(lm + l1 + l2, keepdims=True) / float(_NPL)


def kernel(img1, img1_out, img2, img2_out):
    del img1, img2
    i1f = img1_out.reshape(-1)
    i2f = img2_out.reshape(-1)
    pk = jnp.asarray(_PK_NP)
    gix = jnp.asarray(_GIX_NP)
    lens = jnp.asarray(_LENS_NP)
    chtab = jnp.asarray(_CHTAB_NP)
    (outd,) = _make_sc_kernel()(i1f, i2f, pk, gix, lens, chtab)
    res = pl.pallas_call(
        _epilogue_body,
        out_shape=jax.ShapeDtypeStruct((1,), jnp.float32),
    )(outd)
    return res[0]
